# baseline (speedup 1.0000x reference)
import functools

import jax
import jax.numpy as jnp
from jax import lax
from jax.experimental import pallas as pl
from jax.experimental.pallas import tpu as pltpu
from jax.experimental.pallas import tpu_sc as plsc

NC = 2
NS = 16
CH = 4000


def _round_up(a, b):
  return (a + b - 1) // b * b


def _sc_mesh():
  return plsc.VectorSubcoreMesh(core_axis_name="c", subcore_axis_name="s")


def _make_deg_kernel(T, nchunk):
  @functools.partial(
      pl.kernel,
      out_type=jax.ShapeDtypeStruct((NC, T), jnp.float32),
      mesh=_sc_mesh(),
      scratch_types=[
          pltpu.VMEM((CH,), jnp.int32),
          pltpu.VMEM((CH,), jnp.float32),
          pltpu.VMEM_SHARED((T,), jnp.float32),
      ],
  )
  def deg_kernel(dstv, ones, zeros, out, dst_v, ones_v, acc_sh):
    cid = lax.axis_index("c")
    sid = lax.axis_index("s")
    wid = sid * NC + cid
    rps = T // NS
    sub = pl.ds(sid * rps, rps)
    pltpu.sync_copy(ones, ones_v)
    pltpu.sync_copy(zeros.at[sub], acc_sh.at[sub])
    plsc.subcore_barrier()

    @pl.loop(0, nchunk)
    def _(i):
      off = (wid * nchunk + i) * CH
      pltpu.sync_copy(dstv.at[pl.ds(off, CH)], dst_v)
      pltpu.sync_copy(ones_v, acc_sh.at[dst_v], add=True)

    plsc.subcore_barrier()
    for c in range(NC):
      @pl.when(cid == c)
      def _():
        pltpu.sync_copy(acc_sh.at[sub], out.at[c, sub])

  return deg_kernel


def _make_agg_kernel(T, nchunk, width):
  @functools.partial(
      pl.kernel,
      out_type=jax.ShapeDtypeStruct((NC * width, T), jnp.float32),
      mesh=_sc_mesh(),
      scratch_types=[
          pltpu.VMEM((CH,), jnp.int32),
          pltpu.VMEM((CH,), jnp.int32),
          [pltpu.VMEM((CH,), jnp.float32)] * width,
          [pltpu.VMEM_SHARED((T,), jnp.float32)] * width,
      ],
  )
  def agg_kernel(srcv, dstv, *rest):
    planes = rest[:width]
    zeros = rest[width]
    out = rest[width + 1]
    src_v, dst_v, rows, acc_sh = rest[width + 2:]
    cid = lax.axis_index("c")
    sid = lax.axis_index("s")
    wid = sid * NC + cid
    rps = T // NS
    sub = pl.ds(sid * rps, rps)
    for p in range(width):
      pltpu.sync_copy(zeros.at[sub], acc_sh[p].at[sub])
    plsc.subcore_barrier()

    @pl.loop(0, nchunk)
    def _(i):
      off = (wid * nchunk + i) * CH
      pltpu.sync_copy(srcv.at[pl.ds(off, CH)], src_v)
      pltpu.sync_copy(dstv.at[pl.ds(off, CH)], dst_v)
      for p in range(width):
        pltpu.sync_copy(planes[p].at[src_v], rows[p])
        pltpu.sync_copy(rows[p], acc_sh[p].at[dst_v], add=True)

    plsc.subcore_barrier()
    for c in range(NC):
      @pl.when(cid == c)
      def _():
        for p in range(width):
          pltpu.sync_copy(acc_sh[p].at[sub], out.at[c * width + p, sub])

  return agg_kernel


def _tc1_body(x3t_ref, degp_ref, xs3t_ref, dist_ref):
  deg = degp_ref[0:1, :] + degp_ref[1:2, :] + 1.0
  dis = lax.rsqrt(jnp.maximum(deg, 1.0))
  dist_ref[...] = dis
  xs3t_ref[...] = x3t_ref[...] * dis


def _tc2_body(a0_ref, a1_ref, xs3t_ref, dist_ref, w1t_ref, b1_ref, w2t_ref,
              g2t_ref):
  agg = a0_ref[...] + a1_ref[...] + xs3t_ref[...]
  dis = dist_ref[...]
  h = jnp.dot(w1t_ref[...], agg, preferred_element_type=jnp.float32)
  h = dis * h + b1_ref[...]
  z = jnp.maximum(h, 0.0)
  p = jnp.dot(w2t_ref[...], z, preferred_element_type=jnp.float32)
  g2t_ref[...] = dis * p


def _tc3_body(a0_ref, a1_ref, g2t_ref, dist_ref, b2_ref, out_ref):
  agg = a0_ref[...] + a1_ref[...] + g2t_ref[...]
  out_ref[...] = dist_ref[...] * agg + b2_ref[...]


def kernel(x, edge_index, W1, b1, W2, b2):
  N = x.shape[0]
  E = edge_index.shape[1]
  F1 = W1.shape[1]

  E_pad = _round_up(E, NC * NS * CH)
  nchunk = E_pad // (NC * NS * CH)
  P = E_pad - E
  npad = 512 if P else 0
  T = _round_up(N + npad, 128 * NS)

  if P:
    pad_ar = jnp.arange(P, dtype=jnp.int32)
    srcv = jnp.concatenate([edge_index[0], N + (pad_ar % npad)])
    dstv = jnp.concatenate([edge_index[1], N + ((pad_ar + npad // 2) % npad)])
  else:
    srcv = edge_index[0]
    dstv = edge_index[1]

  x3t = jnp.pad(x, ((0, T - N), (0, 0))).T
  zeros = jnp.zeros((T,), jnp.float32)
  ones = jnp.ones((CH,), jnp.float32)

  degp = _make_deg_kernel(T, nchunk)(dstv, ones, zeros)

  xs3t, dist = pl.pallas_call(
      _tc1_body,
      out_shape=[
          jax.ShapeDtypeStruct((3, T), jnp.float32),
          jax.ShapeDtypeStruct((1, T), jnp.float32),
      ],
  )(x3t, degp)

  acc1f = _make_agg_kernel(T, nchunk, 3)(
      srcv, dstv, xs3t[0], xs3t[1], xs3t[2], zeros)

  g2t, = pl.pallas_call(
      _tc2_body,
      out_shape=[jax.ShapeDtypeStruct((1, T), jnp.float32)],
  )(acc1f[0:3], acc1f[3:6], xs3t, dist, W1.T, b1.reshape(F1, 1), W2.T)

  acc2f = _make_agg_kernel(T, nchunk, 1)(srcv, dstv, g2t.reshape(T), zeros)

  outt, = pl.pallas_call(
      _tc3_body,
      out_shape=[jax.ShapeDtypeStruct((1, T), jnp.float32)],
  )(acc2f[0:1], acc2f[1:2], g2t, dist, b2.reshape(1, 1))

  return outt.reshape(T, 1)[:N]


# speedup vs baseline: 1.7320x; 1.7320x over previous
import functools

import jax
import jax.numpy as jnp
from jax import lax
from jax.experimental import pallas as pl
from jax.experimental.pallas import tpu as pltpu
from jax.experimental.pallas import tpu_sc as plsc

NC = 2
NS = 16
CH = 4000


def _round_up(a, b):
  return (a + b - 1) // b * b


def _sc_mesh():
  return plsc.VectorSubcoreMesh(core_axis_name="c", subcore_axis_name="s")


def _make_deg_kernel(T, nchunk):
  @functools.partial(
      pl.kernel,
      out_type=jax.ShapeDtypeStruct((NC, T), jnp.float32),
      mesh=_sc_mesh(),
      scratch_types=[
          pltpu.VMEM((CH,), jnp.int32),
          pltpu.VMEM((CH,), jnp.float32),
          pltpu.VMEM_SHARED((T,), jnp.float32),
      ],
  )
  def deg_kernel(dstv, ones, zeros, out, dst_v, ones_v, acc_sh):
    cid = lax.axis_index("c")
    sid = lax.axis_index("s")
    wid = sid * NC + cid
    rps = T // NS
    sub = pl.ds(sid * rps, rps)
    pltpu.sync_copy(ones, ones_v)
    pltpu.sync_copy(zeros.at[sub], acc_sh.at[sub])
    plsc.subcore_barrier()

    @pl.loop(0, nchunk)
    def _(i):
      off = (wid * nchunk + i) * CH
      pltpu.sync_copy(dstv.at[pl.ds(off, CH)], dst_v)
      pltpu.sync_copy(ones_v, acc_sh.at[dst_v], add=True)

    plsc.subcore_barrier()
    for c in range(NC):
      @pl.when(cid == c)
      def _():
        pltpu.sync_copy(acc_sh.at[sub], out.at[c, sub])

  return deg_kernel


def _make_agg_kernel(T, nchunk, width):
  @functools.partial(
      pl.kernel,
      out_type=jax.ShapeDtypeStruct((NC * width, T), jnp.float32),
      mesh=_sc_mesh(),
      scratch_types=[
          pltpu.VMEM((CH,), jnp.int32),
          pltpu.VMEM((CH,), jnp.int32),
          [pltpu.VMEM((CH,), jnp.float32)] * width,
          [pltpu.VMEM_SHARED((T,), jnp.float32)] * width,
          [pltpu.VMEM_SHARED((T,), jnp.float32)] * width,
      ],
  )
  def agg_kernel(srcv, dstv, *rest):
    planes = rest[:width]
    zeros = rest[width]
    out = rest[width + 1]
    src_v, dst_v, rows, acc_sh, table_sh = rest[width + 2:]
    cid = lax.axis_index("c")
    sid = lax.axis_index("s")
    wid = sid * NC + cid
    rps = T // NS
    sub = pl.ds(sid * rps, rps)
    for p in range(width):
      pltpu.sync_copy(zeros.at[sub], acc_sh[p].at[sub])
      pltpu.sync_copy(planes[p].at[sub], table_sh[p].at[sub])
    plsc.subcore_barrier()

    @pl.loop(0, nchunk)
    def _(i):
      off = (wid * nchunk + i) * CH
      pltpu.sync_copy(srcv.at[pl.ds(off, CH)], src_v)
      pltpu.sync_copy(dstv.at[pl.ds(off, CH)], dst_v)
      for p in range(width):
        pltpu.sync_copy(table_sh[p].at[src_v], rows[p])
        pltpu.sync_copy(rows[p], acc_sh[p].at[dst_v], add=True)

    plsc.subcore_barrier()
    for c in range(NC):
      @pl.when(cid == c)
      def _():
        for p in range(width):
          pltpu.sync_copy(acc_sh[p].at[sub], out.at[c * width + p, sub])

  return agg_kernel


def _tc1_body(x3t_ref, degp_ref, xs3t_ref, dist_ref):
  deg = degp_ref[0:1, :] + degp_ref[1:2, :] + 1.0
  dis = lax.rsqrt(jnp.maximum(deg, 1.0))
  dist_ref[...] = dis
  xs3t_ref[...] = x3t_ref[...] * dis


def _tc2_body(a0_ref, a1_ref, xs3t_ref, dist_ref, w1t_ref, b1_ref, w2t_ref,
              g2t_ref):
  agg = a0_ref[...] + a1_ref[...] + xs3t_ref[...]
  dis = dist_ref[...]
  h = jnp.dot(w1t_ref[...], agg, preferred_element_type=jnp.float32)
  h = dis * h + b1_ref[...]
  z = jnp.maximum(h, 0.0)
  p = jnp.dot(w2t_ref[...], z, preferred_element_type=jnp.float32)
  g2t_ref[...] = dis * p


def _tc3_body(a0_ref, a1_ref, g2t_ref, dist_ref, b2_ref, out_ref):
  agg = a0_ref[...] + a1_ref[...] + g2t_ref[...]
  out_ref[...] = dist_ref[...] * agg + b2_ref[...]


def kernel(x, edge_index, W1, b1, W2, b2):
  N = x.shape[0]
  E = edge_index.shape[1]
  F1 = W1.shape[1]

  E_pad = _round_up(E, NC * NS * CH)
  nchunk = E_pad // (NC * NS * CH)
  P = E_pad - E
  npad = 512 if P else 0
  T = _round_up(N + npad, 128 * NS)

  if P:
    pad_ar = jnp.arange(P, dtype=jnp.int32)
    srcv = jnp.concatenate([edge_index[0], N + (pad_ar % npad)])
    dstv = jnp.concatenate([edge_index[1], N + ((pad_ar + npad // 2) % npad)])
  else:
    srcv = edge_index[0]
    dstv = edge_index[1]

  x3t = jnp.pad(x, ((0, T - N), (0, 0))).T
  zeros = jnp.zeros((T,), jnp.float32)
  ones = jnp.ones((CH,), jnp.float32)

  degp = _make_deg_kernel(T, nchunk)(dstv, ones, zeros)

  xs3t, dist = pl.pallas_call(
      _tc1_body,
      out_shape=[
          jax.ShapeDtypeStruct((3, T), jnp.float32),
          jax.ShapeDtypeStruct((1, T), jnp.float32),
      ],
  )(x3t, degp)

  acc1f = _make_agg_kernel(T, nchunk, 3)(
      srcv, dstv, xs3t[0], xs3t[1], xs3t[2], zeros)

  g2t, = pl.pallas_call(
      _tc2_body,
      out_shape=[jax.ShapeDtypeStruct((1, T), jnp.float32)],
  )(acc1f[0:3], acc1f[3:6], xs3t, dist, W1.T, b1.reshape(F1, 1), W2.T)

  acc2f = _make_agg_kernel(T, nchunk, 1)(srcv, dstv, g2t.reshape(T), zeros)

  outt, = pl.pallas_call(
      _tc3_body,
      out_shape=[jax.ShapeDtypeStruct((1, T), jnp.float32)],
  )(acc2f[0:1], acc2f[1:2], g2t, dist, b2.reshape(1, 1))

  return outt.reshape(T, 1)[:N]


# speedup vs baseline: 1.8439x; 1.0646x over previous
import functools

import jax
import jax.numpy as jnp
from jax import lax
from jax.experimental import pallas as pl
from jax.experimental.pallas import tpu as pltpu
from jax.experimental.pallas import tpu_sc as plsc

NC = 2
NS = 16
CH = 4000


def _round_up(a, b):
  return (a + b - 1) // b * b


def _sc_mesh():
  return plsc.VectorSubcoreMesh(core_axis_name="c", subcore_axis_name="s")


def _make_deg_kernel(T, nchunk):
  @functools.partial(
      pl.kernel,
      out_type=jax.ShapeDtypeStruct((NC, T), jnp.float32),
      mesh=_sc_mesh(),
      scratch_types=[
          pltpu.VMEM((CH,), jnp.int32),
          pltpu.VMEM((CH,), jnp.float32),
          pltpu.VMEM_SHARED((T,), jnp.float32),
      ],
  )
  def deg_kernel(dstv, ones, zeros, out, dst_v, ones_v, acc_sh):
    cid = lax.axis_index("c")
    sid = lax.axis_index("s")
    wid = sid * NC + cid
    rps = T // NS
    sub = pl.ds(sid * rps, rps)
    pltpu.sync_copy(ones, ones_v)
    pltpu.sync_copy(zeros.at[sub], acc_sh.at[sub])
    plsc.subcore_barrier()

    @pl.loop(0, nchunk)
    def _(i):
      off = (wid * nchunk + i) * CH
      pltpu.sync_copy(dstv.at[pl.ds(off, CH)], dst_v)
      pltpu.sync_copy(ones_v, acc_sh.at[dst_v], add=True)

    plsc.subcore_barrier()
    for c in range(NC):
      @pl.when(cid == c)
      def _():
        pltpu.sync_copy(acc_sh.at[sub], out.at[c, sub])

  return deg_kernel


def _make_agg_kernel(T, nchunk, width):
  @functools.partial(
      pl.kernel,
      out_type=jax.ShapeDtypeStruct((NC * width, T), jnp.float32),
      mesh=_sc_mesh(),
      scratch_types=[
          pltpu.VMEM((CH,), jnp.int32),
          pltpu.VMEM((CH,), jnp.int32),
          [pltpu.VMEM((CH,), jnp.float32)] * width,
          [pltpu.VMEM_SHARED((T,), jnp.float32)] * width,
          [pltpu.VMEM_SHARED((T,), jnp.float32)] * width,
          [pltpu.SemaphoreType.DMA] * width,
          [pltpu.SemaphoreType.DMA] * width,
      ],
  )
  def agg_kernel(srcv, dstv, *rest):
    planes = rest[:width]
    zeros = rest[width]
    out = rest[width + 1]
    src_v, dst_v, rows, acc_sh, table_sh, gsem, ssem = rest[width + 2:]
    cid = lax.axis_index("c")
    sid = lax.axis_index("s")
    wid = sid * NC + cid
    rps = T // NS
    sub = pl.ds(sid * rps, rps)
    for p in range(width):
      pltpu.sync_copy(zeros.at[sub], acc_sh[p].at[sub])
      pltpu.sync_copy(planes[p].at[sub], table_sh[p].at[sub])
    plsc.subcore_barrier()

    @pl.loop(0, nchunk)
    def _(i):
      off = (wid * nchunk + i) * CH
      pltpu.sync_copy(srcv.at[pl.ds(off, CH)], src_v)
      pltpu.sync_copy(dstv.at[pl.ds(off, CH)], dst_v)
      gd = [pltpu.async_copy(table_sh[p].at[src_v], rows[p], gsem[p])
            for p in range(width)]
      sd = []
      for p in range(width):
        gd[p].wait()
        sd.append(
            pltpu.async_copy(rows[p], acc_sh[p].at[dst_v], ssem[p], add=True))
      for p in range(width):
        sd[p].wait()

    plsc.subcore_barrier()
    for c in range(NC):
      @pl.when(cid == c)
      def _():
        for p in range(width):
          pltpu.sync_copy(acc_sh[p].at[sub], out.at[c * width + p, sub])

  return agg_kernel


def _tc1_body(x3t_ref, degp_ref, xs3t_ref, dist_ref):
  deg = degp_ref[0:1, :] + degp_ref[1:2, :] + 1.0
  dis = lax.rsqrt(jnp.maximum(deg, 1.0))
  dist_ref[...] = dis
  xs3t_ref[...] = x3t_ref[...] * dis


def _tc2_body(a0_ref, a1_ref, xs3t_ref, dist_ref, w1t_ref, b1_ref, w2t_ref,
              g2t_ref):
  agg = a0_ref[...] + a1_ref[...] + xs3t_ref[...]
  dis = dist_ref[...]
  h = jnp.dot(w1t_ref[...], agg, preferred_element_type=jnp.float32)
  h = dis * h + b1_ref[...]
  z = jnp.maximum(h, 0.0)
  p = jnp.dot(w2t_ref[...], z, preferred_element_type=jnp.float32)
  g2t_ref[...] = dis * p


def _tc3_body(a0_ref, a1_ref, g2t_ref, dist_ref, b2_ref, out_ref):
  agg = a0_ref[...] + a1_ref[...] + g2t_ref[...]
  out_ref[...] = dist_ref[...] * agg + b2_ref[...]


def kernel(x, edge_index, W1, b1, W2, b2):
  N = x.shape[0]
  E = edge_index.shape[1]
  F1 = W1.shape[1]

  E_pad = _round_up(E, NC * NS * CH)
  nchunk = E_pad // (NC * NS * CH)
  P = E_pad - E
  npad = 512 if P else 0
  T = _round_up(N + npad, 128 * NS)

  if P:
    pad_ar = jnp.arange(P, dtype=jnp.int32)
    srcv = jnp.concatenate([edge_index[0], N + (pad_ar % npad)])
    dstv = jnp.concatenate([edge_index[1], N + ((pad_ar + npad // 2) % npad)])
  else:
    srcv = edge_index[0]
    dstv = edge_index[1]

  x3t = jnp.pad(x, ((0, T - N), (0, 0))).T
  zeros = jnp.zeros((T,), jnp.float32)
  ones = jnp.ones((CH,), jnp.float32)

  degp = _make_deg_kernel(T, nchunk)(dstv, ones, zeros)

  xs3t, dist = pl.pallas_call(
      _tc1_body,
      out_shape=[
          jax.ShapeDtypeStruct((3, T), jnp.float32),
          jax.ShapeDtypeStruct((1, T), jnp.float32),
      ],
  )(x3t, degp)

  acc1f = _make_agg_kernel(T, nchunk, 3)(
      srcv, dstv, xs3t[0], xs3t[1], xs3t[2], zeros)

  g2t, = pl.pallas_call(
      _tc2_body,
      out_shape=[jax.ShapeDtypeStruct((1, T), jnp.float32)],
  )(acc1f[0:3], acc1f[3:6], xs3t, dist, W1.T, b1.reshape(F1, 1), W2.T)

  acc2f = _make_agg_kernel(T, nchunk, 1)(srcv, dstv, g2t.reshape(T), zeros)

  outt, = pl.pallas_call(
      _tc3_body,
      out_shape=[jax.ShapeDtypeStruct((1, T), jnp.float32)],
  )(acc2f[0:1], acc2f[1:2], g2t, dist, b2.reshape(1, 1))

  return outt.reshape(T, 1)[:N]
